# baseline (speedup 1.0000x reference)
import functools

import jax
import jax.numpy as jnp
from jax import lax
from jax.experimental import pallas as pl
from jax.experimental.pallas import tpu as pltpu
from jax.experimental.pallas import tpu_sc as plsc

VOCAB = 1000000
EMBED = 64
HIDDEN = 256
NCLASS = 16
BATCH = 16384
HIST = 50

_NC = 2
_NS = 16
NW = _NC * _NS
BPW = BATCH // NW
LANES = 16
EV = EMBED // LANES


BAGS_PER_DMA = 2
NPAIR = BPW // BAGS_PER_DMA
NBUF = 8


def _sc_pooled(table, data3):
    mesh = plsc.VectorSubcoreMesh(core_axis_name="c", subcore_axis_name="s")
    GROUP = BAGS_PER_DMA * HIST

    HIMASK = jnp.int32(-65536)

    @functools.partial(
        pl.kernel,
        mesh=mesh,
        out_type=jax.ShapeDtypeStruct((BATCH, EMBED), jnp.float32),
        scratch_types=[
            pltpu.VMEM((NPAIR, GROUP), jnp.int32),
            pltpu.VMEM((NBUF, GROUP, EMBED // 2), jnp.int32),
            pltpu.VMEM((BPW, EMBED), jnp.float32),
            [pltpu.SemaphoreType.DMA] * NBUF,
        ],
        compiler_params=pltpu.CompilerParams(
            use_tc_tiling_on_sc=False, needs_layout_passes=False
        ),
    )
    def sc_kernel(table_hbm, data_hbm, out_hbm, idx_v, rows_v, pooled_v, sems):
        wid = lax.axis_index("s") * _NC + lax.axis_index("c")
        pltpu.sync_copy(data_hbm.at[wid], idx_v)

        for b in range(NBUF):
            pltpu.async_copy(table_hbm.at[idx_v.at[b]], rows_v.at[b], sems[b])

        def split_pair(w):
            lo = plsc.bitcast(w << 16, jnp.float32)
            hi = plsc.bitcast(w & HIMASK, jnp.float32)
            return lo, hi

        def group_body(i, _):
            p0 = i * NBUF
            for b in range(NBUF):
                p = p0 + b
                buf = rows_v.at[b]
                pltpu.make_async_copy(table_hbm.at[idx_v.at[p]], buf, sems[b]).wait()
                for bag in range(BAGS_PER_DMA):
                    base = bag * HIST
                    accs = [None] * EV
                    for j in range(HIST):
                        for g in range(EV // 2):
                            w = buf[base + j, pl.ds(LANES * g, LANES)]
                            lo, hi = split_pair(w)
                            if j == 0:
                                accs[g], accs[g + 2] = lo, hi
                            else:
                                accs[g] = accs[g] + lo
                                accs[g + 2] = accs[g + 2] + hi
                    for k in range(EV):
                        pooled_v[p * BAGS_PER_DMA + bag, pl.ds(LANES * k, LANES)] = (
                            accs[k] * (1.0 / HIST)
                        )
                nxt = p + NBUF

                @pl.when(nxt < NPAIR)
                def _():
                    pltpu.async_copy(table_hbm.at[idx_v.at[nxt]], buf, sems[b])

            return 0

        lax.fori_loop(0, NPAIR // NBUF, group_body, 0)
        pltpu.sync_copy(pooled_v, out_hbm.at[pl.ds(wid * BPW, BPW)])

    return sc_kernel(table, data3)


def _tc_mlp(pooled, W1, b1, W2, b2):
    BLK = 4096

    def body(x_ref, w1_ref, b1_ref, w2_ref, b2_ref, o_ref):
        x = x_ref[...]
        h = jnp.dot(x, w1_ref[...], preferred_element_type=jnp.float32) + b1_ref[...]
        l = jnp.dot(h, w2_ref[...], preferred_element_type=jnp.float32) + b2_ref[...]
        m = jnp.max(l, axis=-1, keepdims=True)
        e = jnp.exp(l - m)
        o_ref[...] = e / jnp.sum(e, axis=-1, keepdims=True)

    return pl.pallas_call(
        body,
        grid=(BATCH // BLK,),
        in_specs=[
            pl.BlockSpec((BLK, EMBED), lambda i: (i, 0)),
            pl.BlockSpec((EMBED, HIDDEN), lambda i: (0, 0)),
            pl.BlockSpec((1, HIDDEN), lambda i: (0, 0)),
            pl.BlockSpec((HIDDEN, NCLASS), lambda i: (0, 0)),
            pl.BlockSpec((1, NCLASS), lambda i: (0, 0)),
        ],
        out_specs=pl.BlockSpec((BLK, NCLASS), lambda i: (i, 0)),
        out_shape=jax.ShapeDtypeStruct((BATCH, NCLASS), jnp.float32),
    )(pooled, W1, b1.reshape(1, HIDDEN), W2, b2.reshape(1, NCLASS))


VB = 32768
VGRID = -(-VOCAB // VB)
VPAD = VGRID * VB


def _tc_detile(tableT):
    def body(x_ref, o_ref):
        x = x_ref[...]
        abits = jax.lax.bitcast_convert_type(
            x[: EMBED // 2, :].astype(jnp.bfloat16).astype(jnp.float32), jnp.int32
        )
        bbits = jax.lax.bitcast_convert_type(
            x[EMBED // 2 :, :].astype(jnp.bfloat16).astype(jnp.float32), jnp.int32
        )
        ab = jax.lax.shift_right_logical(abits, 16) | (bbits & jnp.int32(-65536))
        q = VB // 4
        c = jnp.concatenate([ab[:, h * q : (h + 1) * q] for h in range(4)], axis=0)
        o_ref[...] = c.T

    return pl.pallas_call(
        body,
        grid=(VGRID,),
        in_specs=[pl.BlockSpec((EMBED, VB), lambda i: (0, i))],
        out_specs=pl.BlockSpec((VB // 4, 2 * EMBED), lambda i: (i, 0)),
        out_shape=jax.ShapeDtypeStruct((VPAD // 4, 2 * EMBED), jnp.int32),
    )(tableT)


def kernel(data, table, W1, b1, W2, b2):
    v = data.astype(jnp.int32)
    r = v & (VB - 1)
    vmap = (v - r) + ((r & (VB // 4 - 1)) << 2) + (r >> 13)
    data3 = vmap.reshape(NW, NPAIR, BAGS_PER_DMA * HIST)
    table_lin = _tc_detile(table.T).reshape(VPAD, EMBED // 2)
    pooled = _sc_pooled(table_lin, data3)
    return _tc_mlp(pooled, W1, b1, W2, b2)


# speedup vs baseline: 1.0757x; 1.0757x over previous
import functools

import jax
import jax.numpy as jnp
from jax import lax
from jax.experimental import pallas as pl
from jax.experimental.pallas import tpu as pltpu
from jax.experimental.pallas import tpu_sc as plsc

VOCAB = 1000000
EMBED = 64
HIDDEN = 256
NCLASS = 16
BATCH = 16384
HIST = 50

_NC = 2
_NS = 16
NW = _NC * _NS
BPW = BATCH // NW
LANES = 16
EV = EMBED // LANES


BAGS_PER_DMA = 1
NPAIR = BPW // BAGS_PER_DMA
NBUF = 4


def _sc_pooled(table, data3):
    mesh = plsc.VectorSubcoreMesh(core_axis_name="c", subcore_axis_name="s")
    GROUP = BAGS_PER_DMA * HIST

    HIMASK = jnp.int32(-65536)

    @functools.partial(
        pl.kernel,
        mesh=mesh,
        out_type=jax.ShapeDtypeStruct((BATCH, EMBED), jnp.float32),
        scratch_types=[
            pltpu.VMEM((NPAIR, GROUP), jnp.int32),
            pltpu.VMEM((NBUF, GROUP, EMBED // 2), jnp.int32),
            pltpu.VMEM((BPW, EMBED), jnp.float32),
            [pltpu.SemaphoreType.DMA] * NBUF,
        ],
        compiler_params=pltpu.CompilerParams(
            use_tc_tiling_on_sc=False, needs_layout_passes=False
        ),
    )
    def sc_kernel(table_hbm, data_hbm, out_hbm, idx_v, rows_v, pooled_v, sems):
        wid = lax.axis_index("s") * _NC + lax.axis_index("c")
        pltpu.sync_copy(data_hbm.at[wid], idx_v)

        for b in range(NBUF):
            pltpu.async_copy(table_hbm.at[idx_v.at[b]], rows_v.at[b], sems[b])

        def split_pair(w):
            lo = plsc.bitcast(w << 16, jnp.float32)
            hi = plsc.bitcast(w & HIMASK, jnp.float32)
            return lo, hi

        def group_body(i, _):
            p0 = i * NBUF
            for b in range(NBUF):
                p = p0 + b
                buf = rows_v.at[b]
                pltpu.make_async_copy(table_hbm.at[idx_v.at[p]], buf, sems[b]).wait()
                for bag in range(BAGS_PER_DMA):
                    base = bag * HIST
                    accs = [None] * EV
                    for j in range(HIST):
                        for g in range(EV // 2):
                            w = buf[base + j, pl.ds(LANES * g, LANES)]
                            lo, hi = split_pair(w)
                            if j == 0:
                                accs[g], accs[g + 2] = lo, hi
                            else:
                                accs[g] = accs[g] + lo
                                accs[g + 2] = accs[g + 2] + hi
                    for k in range(EV):
                        pooled_v[p * BAGS_PER_DMA + bag, pl.ds(LANES * k, LANES)] = (
                            accs[k] * (1.0 / HIST)
                        )
                nxt = p + NBUF

                @pl.when(nxt < NPAIR)
                def _():
                    pltpu.async_copy(table_hbm.at[idx_v.at[nxt]], buf, sems[b])

            return 0

        lax.fori_loop(0, NPAIR // NBUF, group_body, 0)
        pltpu.sync_copy(pooled_v, out_hbm.at[pl.ds(wid * BPW, BPW)])

    return sc_kernel(table, data3)


def _tc_mlp(pooled, W1, b1, W2, b2):
    BLK = 4096

    def body(x_ref, w1_ref, b1_ref, w2_ref, b2_ref, o_ref):
        x = x_ref[...]
        h = jnp.dot(x, w1_ref[...], preferred_element_type=jnp.float32) + b1_ref[...]
        l = jnp.dot(h, w2_ref[...], preferred_element_type=jnp.float32) + b2_ref[...]
        m = jnp.max(l, axis=-1, keepdims=True)
        e = jnp.exp(l - m)
        o_ref[...] = e / jnp.sum(e, axis=-1, keepdims=True)

    return pl.pallas_call(
        body,
        grid=(BATCH // BLK,),
        in_specs=[
            pl.BlockSpec((BLK, EMBED), lambda i: (i, 0)),
            pl.BlockSpec((EMBED, HIDDEN), lambda i: (0, 0)),
            pl.BlockSpec((1, HIDDEN), lambda i: (0, 0)),
            pl.BlockSpec((HIDDEN, NCLASS), lambda i: (0, 0)),
            pl.BlockSpec((1, NCLASS), lambda i: (0, 0)),
        ],
        out_specs=pl.BlockSpec((BLK, NCLASS), lambda i: (i, 0)),
        out_shape=jax.ShapeDtypeStruct((BATCH, NCLASS), jnp.float32),
    )(pooled, W1, b1.reshape(1, HIDDEN), W2, b2.reshape(1, NCLASS))


VB = 32768
VGRID = -(-VOCAB // VB)
VPAD = VGRID * VB


def _tc_detile(tableT):
    def body(x_ref, o_ref):
        x = x_ref[...]
        abits = jax.lax.bitcast_convert_type(
            x[: EMBED // 2, :].astype(jnp.bfloat16).astype(jnp.float32), jnp.int32
        )
        bbits = jax.lax.bitcast_convert_type(
            x[EMBED // 2 :, :].astype(jnp.bfloat16).astype(jnp.float32), jnp.int32
        )
        ab = jax.lax.shift_right_logical(abits, 16) | (bbits & jnp.int32(-65536))
        q = VB // 4
        c = jnp.concatenate([ab[:, h * q : (h + 1) * q] for h in range(4)], axis=0)
        o_ref[...] = c.T

    return pl.pallas_call(
        body,
        grid=(VGRID,),
        in_specs=[pl.BlockSpec((EMBED, VB), lambda i: (0, i))],
        out_specs=pl.BlockSpec((VB // 4, 2 * EMBED), lambda i: (i, 0)),
        out_shape=jax.ShapeDtypeStruct((VPAD // 4, 2 * EMBED), jnp.int32),
    )(tableT)


def kernel(data, table, W1, b1, W2, b2):
    v = data.astype(jnp.int32)
    r = v & (VB - 1)
    vmap = (v - r) + ((r & (VB // 4 - 1)) << 2) + (r >> 13)
    data3 = vmap.reshape(NW, NPAIR, BAGS_PER_DMA * HIST)
    table_lin = _tc_detile(table.T).reshape(VPAD, EMBED // 2)
    pooled = _sc_pooled(table_lin, data3)
    return _tc_mlp(pooled, W1, b1, W2, b2)


# speedup vs baseline: 1.2353x; 1.1484x over previous
import functools

import jax
import jax.numpy as jnp
from jax import lax
from jax.experimental import pallas as pl
from jax.experimental.pallas import tpu as pltpu
from jax.experimental.pallas import tpu_sc as plsc

VOCAB = 1000000
EMBED = 64
HIDDEN = 256
NCLASS = 16
BATCH = 16384
HIST = 50

_NC = 2
_NS = 16
NW = _NC * _NS
BPW = BATCH // NW
LANES = 16
EV = EMBED // LANES


BAGS_PER_DMA = 2
NPAIR = BPW // BAGS_PER_DMA
NBUF = 4


def _sc_pooled(table, data3):
    mesh = plsc.VectorSubcoreMesh(core_axis_name="c", subcore_axis_name="s")
    GROUP = BAGS_PER_DMA * HIST

    HIMASK = jnp.int32(-65536)

    @functools.partial(
        pl.kernel,
        mesh=mesh,
        out_type=jax.ShapeDtypeStruct((BATCH, EMBED), jnp.float32),
        scratch_types=[
            pltpu.VMEM((NPAIR, GROUP), jnp.int32),
            pltpu.VMEM((NBUF, GROUP, EMBED // 2), jnp.int32),
            pltpu.VMEM((BPW, EMBED), jnp.float32),
            [pltpu.SemaphoreType.DMA] * NBUF,
        ],
        compiler_params=pltpu.CompilerParams(
            use_tc_tiling_on_sc=False, needs_layout_passes=False
        ),
    )
    def sc_kernel(table_hbm, data_hbm, out_hbm, idx_v, rows_v, pooled_v, sems):
        wid = lax.axis_index("s") * _NC + lax.axis_index("c")
        pltpu.sync_copy(data_hbm.at[wid], idx_v)

        for b in range(NBUF):
            pltpu.async_copy(table_hbm.at[idx_v.at[b]], rows_v.at[b], sems[b])

        def split_pair(w):
            lo = plsc.bitcast(w << 16, jnp.float32)
            hi = plsc.bitcast(w & HIMASK, jnp.float32)
            return lo, hi

        def group_body(i, _):
            p0 = i * NBUF
            for b in range(NBUF):
                p = p0 + b
                buf = rows_v.at[b]
                pltpu.make_async_copy(table_hbm.at[idx_v.at[p]], buf, sems[b]).wait()
                for bag in range(BAGS_PER_DMA):
                    base = bag * HIST
                    accs = [None] * EV
                    for j in range(HIST):
                        for g in range(EV // 2):
                            w = buf[base + j, pl.ds(LANES * g, LANES)]
                            lo, hi = split_pair(w)
                            if j == 0:
                                accs[g], accs[g + 2] = lo, hi
                            else:
                                accs[g] = accs[g] + lo
                                accs[g + 2] = accs[g + 2] + hi
                    for k in range(EV):
                        pooled_v[p * BAGS_PER_DMA + bag, pl.ds(LANES * k, LANES)] = (
                            accs[k] * (1.0 / HIST)
                        )
                nxt = p + NBUF

                @pl.when(nxt < NPAIR)
                def _():
                    pltpu.async_copy(table_hbm.at[idx_v.at[nxt]], buf, sems[b])

            return 0

        lax.fori_loop(0, NPAIR // NBUF, group_body, 0)
        pltpu.sync_copy(pooled_v, out_hbm.at[pl.ds(wid * BPW, BPW)])

    return sc_kernel(table, data3)


def _tc_mlp(pooled, W1, b1, W2, b2):
    BLK = 4096

    def body(x_ref, w1_ref, b1_ref, w2_ref, b2_ref, o_ref):
        x = x_ref[...]
        h = jnp.dot(x, w1_ref[...], preferred_element_type=jnp.float32) + b1_ref[...]
        l = jnp.dot(h, w2_ref[...], preferred_element_type=jnp.float32) + b2_ref[...]
        m = jnp.max(l, axis=-1, keepdims=True)
        e = jnp.exp(l - m)
        o_ref[...] = e / jnp.sum(e, axis=-1, keepdims=True)

    return pl.pallas_call(
        body,
        grid=(BATCH // BLK,),
        in_specs=[
            pl.BlockSpec((BLK, EMBED), lambda i: (i, 0)),
            pl.BlockSpec((EMBED, HIDDEN), lambda i: (0, 0)),
            pl.BlockSpec((1, HIDDEN), lambda i: (0, 0)),
            pl.BlockSpec((HIDDEN, NCLASS), lambda i: (0, 0)),
            pl.BlockSpec((1, NCLASS), lambda i: (0, 0)),
        ],
        out_specs=pl.BlockSpec((BLK, NCLASS), lambda i: (i, 0)),
        out_shape=jax.ShapeDtypeStruct((BATCH, NCLASS), jnp.float32),
    )(pooled, W1, b1.reshape(1, HIDDEN), W2, b2.reshape(1, NCLASS))


VB = 65536
VGRID = -(-VOCAB // VB)
VPAD = VGRID * VB


def _tc_detile(tableT):
    def body(x_ref, o_ref):
        x = x_ref[...]
        abits = jax.lax.bitcast_convert_type(
            x[: EMBED // 2, :].astype(jnp.bfloat16).astype(jnp.float32), jnp.int32
        )
        bbits = jax.lax.bitcast_convert_type(
            x[EMBED // 2 :, :].astype(jnp.bfloat16).astype(jnp.float32), jnp.int32
        )
        ab = jax.lax.shift_right_logical(abits, 16) | (bbits & jnp.int32(-65536))
        q = VB // 4
        c = jnp.concatenate([ab[:, h * q : (h + 1) * q] for h in range(4)], axis=0)
        o_ref[...] = c.T

    return pl.pallas_call(
        body,
        grid=(VGRID,),
        in_specs=[pl.BlockSpec((EMBED, VB), lambda i: (0, i))],
        out_specs=pl.BlockSpec((VB // 4, 2 * EMBED), lambda i: (i, 0)),
        out_shape=jax.ShapeDtypeStruct((VPAD // 4, 2 * EMBED), jnp.int32),
    )(tableT)


def kernel(data, table, W1, b1, W2, b2):
    v = data.astype(jnp.int32)
    r = v & (VB - 1)
    vmap = (v - r) + ((r & (VB // 4 - 1)) << 2) + (r >> 14)
    data3 = vmap.reshape(NW, NPAIR, BAGS_PER_DMA * HIST)
    table_lin = _tc_detile(table.T).reshape(VPAD, EMBED // 2)
    pooled = _sc_pooled(table_lin, data3)
    return _tc_mlp(pooled, W1, b1, W2, b2)


# speedup vs baseline: 1.2385x; 1.0026x over previous
import functools

import jax
import jax.numpy as jnp
from jax import lax
from jax.experimental import pallas as pl
from jax.experimental.pallas import tpu as pltpu
from jax.experimental.pallas import tpu_sc as plsc

VOCAB = 1000000
EMBED = 64
HIDDEN = 256
NCLASS = 16
BATCH = 16384
HIST = 50

_NC = 2
_NS = 16
NW = _NC * _NS
BPW = BATCH // NW
LANES = 16
EV = EMBED // LANES


BAGS_PER_DMA = 2
NPAIR = BPW // BAGS_PER_DMA
NBUF = 4


def _sc_pooled(table, data3):
    mesh = plsc.VectorSubcoreMesh(core_axis_name="c", subcore_axis_name="s")
    GROUP = BAGS_PER_DMA * HIST

    HIMASK = jnp.int32(-65536)

    @functools.partial(
        pl.kernel,
        mesh=mesh,
        out_type=jax.ShapeDtypeStruct((BATCH, EMBED), jnp.float32),
        scratch_types=[
            pltpu.VMEM((NPAIR, GROUP), jnp.int32),
            pltpu.VMEM((NBUF, GROUP, EMBED // 2), jnp.int32),
            pltpu.VMEM((BPW, EMBED), jnp.float32),
            [pltpu.SemaphoreType.DMA] * NBUF,
        ],
        compiler_params=pltpu.CompilerParams(
            use_tc_tiling_on_sc=False, needs_layout_passes=False
        ),
    )
    def sc_kernel(table_hbm, data_hbm, out_hbm, idx_v, rows_v, pooled_v, sems):
        wid = lax.axis_index("s") * _NC + lax.axis_index("c")
        pltpu.sync_copy(data_hbm.at[wid], idx_v)

        for b in range(NBUF):
            pltpu.async_copy(table_hbm.at[idx_v.at[b]], rows_v.at[b], sems[b])

        def split_pair(w):
            lo = plsc.bitcast(w << 16, jnp.float32)
            hi = plsc.bitcast(w & HIMASK, jnp.float32)
            return lo, hi

        def group_body(i, _):
            p0 = i * NBUF
            for b in range(NBUF):
                p = p0 + b
                buf = rows_v.at[b]
                pltpu.make_async_copy(table_hbm.at[idx_v.at[p]], buf, sems[b]).wait()
                for bag in range(BAGS_PER_DMA):
                    base = bag * HIST
                    accs = [None] * EV
                    for j in range(HIST):
                        for g in range(EV // 2):
                            w = buf[base + j, pl.ds(LANES * g, LANES)]
                            lo, hi = split_pair(w)
                            if j == 0:
                                accs[g], accs[g + 2] = lo, hi
                            else:
                                accs[g] = accs[g] + lo
                                accs[g + 2] = accs[g + 2] + hi
                    for k in range(EV):
                        pooled_v[p * BAGS_PER_DMA + bag, pl.ds(LANES * k, LANES)] = (
                            accs[k] * (1.0 / HIST)
                        )
                nxt = p + NBUF

                @pl.when(nxt < NPAIR)
                def _():
                    pltpu.async_copy(table_hbm.at[idx_v.at[nxt]], buf, sems[b])

            return 0

        lax.fori_loop(0, NPAIR // NBUF, group_body, 0)
        pltpu.sync_copy(pooled_v, out_hbm.at[pl.ds(wid * BPW, BPW)])

    return sc_kernel(table, data3)


def _tc_mlp(pooled, W1, b1, W2, b2):
    BLK = 4096

    def body(x_ref, w1_ref, b1_ref, w2_ref, b2_ref, o_ref):
        x = x_ref[...]
        h = jnp.dot(x, w1_ref[...], preferred_element_type=jnp.float32) + b1_ref[...]
        l = jnp.dot(h, w2_ref[...], preferred_element_type=jnp.float32) + b2_ref[...]
        m = jnp.max(l, axis=-1, keepdims=True)
        e = jnp.exp(l - m)
        o_ref[...] = e / jnp.sum(e, axis=-1, keepdims=True)

    return pl.pallas_call(
        body,
        grid=(BATCH // BLK,),
        in_specs=[
            pl.BlockSpec((BLK, EMBED), lambda i: (i, 0)),
            pl.BlockSpec((EMBED, HIDDEN), lambda i: (0, 0)),
            pl.BlockSpec((1, HIDDEN), lambda i: (0, 0)),
            pl.BlockSpec((HIDDEN, NCLASS), lambda i: (0, 0)),
            pl.BlockSpec((1, NCLASS), lambda i: (0, 0)),
        ],
        out_specs=pl.BlockSpec((BLK, NCLASS), lambda i: (i, 0)),
        out_shape=jax.ShapeDtypeStruct((BATCH, NCLASS), jnp.float32),
    )(pooled, W1, b1.reshape(1, HIDDEN), W2, b2.reshape(1, NCLASS))


VB = 32768
VGRID = -(-VOCAB // VB)
VPAD = VGRID * VB


def _tc_detile(tableT):
    def body(x_ref, o_ref):
        x = x_ref[...]
        abits = jax.lax.bitcast_convert_type(
            x[: EMBED // 2, :].astype(jnp.bfloat16).astype(jnp.float32), jnp.int32
        )
        bbits = jax.lax.bitcast_convert_type(
            x[EMBED // 2 :, :].astype(jnp.bfloat16).astype(jnp.float32), jnp.int32
        )
        ab = jax.lax.shift_right_logical(abits, 16) | (bbits & jnp.int32(-65536))
        q = VB // 4
        c = jnp.concatenate([ab[:, h * q : (h + 1) * q] for h in range(4)], axis=0)
        o_ref[...] = c.T

    return pl.pallas_call(
        body,
        grid=(VGRID,),
        in_specs=[pl.BlockSpec((EMBED, VB), lambda i: (0, i))],
        out_specs=pl.BlockSpec((VB // 4, 2 * EMBED), lambda i: (i, 0)),
        out_shape=jax.ShapeDtypeStruct((VPAD // 4, 2 * EMBED), jnp.int32),
    )(tableT)


def kernel(data, table, W1, b1, W2, b2):
    v = data.astype(jnp.int32)
    r = v & (VB - 1)
    vmap = (v - r) + ((r & (VB // 4 - 1)) << 2) + (r >> 13)
    data3 = vmap.reshape(NW, NPAIR, BAGS_PER_DMA * HIST)
    table_lin = _tc_detile(table.T).reshape(VPAD, EMBED // 2)
    pooled = _sc_pooled(table_lin, data3)
    return _tc_mlp(pooled, W1, b1, W2, b2)
